# baseline (speedup 1.0000x reference)
import dataclasses
import functools

import jax
import jax.numpy as jnp
from jax import lax
from jax.experimental import pallas as pl
from jax.experimental.pallas import tpu as pltpu
from jax.experimental.pallas import tpu_sc as plsc

N = 10000
D = 128
E = 320000
HOPS = 3

NC = 2
NS = 16
LANES = 16

W = 256
NW = E // W
WIN_PER_TILE = 2 * (-(-(-(-NW // NS)) // 2))
GCHUNK = 32
MAXCH = W // GCHUNK
PACK_BITS = 14
PACK_MASK = (1 << PACK_BITS) - 1
HALF_T = HOPS * N // NC
TRASH = HALF_T
ACC_ROWS = 15104
WB_ROWS = ACC_ROWS // NS


def _sc_body(x_hbm, src_hbm, tgt_hbm, out_hbm,
             srcb, tgtb, pidxc, gidx3, sidx3, rows0, rows1,
             sem_i0, sem_i1, sem_g0, sem_g1, sem_s0, sem_s1, acc):
    c = lax.axis_index("c")
    tid = lax.axis_index("s")
    lo = c * HALF_T
    rows = (rows0, rows1)
    sem_i = (sem_i0, sem_i1)
    sem_g = (sem_g0, sem_g1)
    sem_s = (sem_s0, sem_s1)

    @pl.loop(0, GCHUNK)
    def _(i):
        @pl.loop(0, D, step=LANES)
        def _(j):
            rows0[i, pl.ds(j, LANES)] = jnp.zeros((LANES,), jnp.float32)

    zbase = tid * WB_ROWS
    NZ = WB_ROWS // GCHUNK

    @pl.loop(0, NZ)
    def _(i):
        pltpu.sync_copy(rows0, acc.at[pl.ds(zbase + i * GCHUNK, GCHUNK), :])

    pltpu.sync_copy(rows0.at[pl.ds(0, WB_ROWS - NZ * GCHUNK), :],
                    acc.at[pl.ds(zbase + NZ * GCHUNK, WB_ROWS - NZ * GCHUNK), :])

    plsc.subcore_barrier()

    def idx_base(k):
        w = tid + k * NS
        return jnp.where(w < NW, w * W, 0)

    def start_idx_load(k, p):
        b = idx_base(k)
        pltpu.async_copy(src_hbm.at[pl.ds(b, W)], srcb.at[p], sem_i[p])
        pltpu.async_copy(tgt_hbm.at[pl.ds(b, W)], tgtb.at[p], sem_i[p])

    def wait_idx_load(k, p):
        b = idx_base(k)
        pltpu.make_async_copy(src_hbm.at[pl.ds(b, W)], srcb.at[p],
                              sem_i[p]).wait()
        pltpu.make_async_copy(tgt_hbm.at[pl.ds(b, W)], tgtb.at[p],
                              sem_i[p]).wait()

    def run_window(k, p):
        wait_idx_load(k, p)

        @pl.when(k + 1 < WIN_PER_TILE)
        def _():
            start_idx_load(k + 1, 1 - p)

        valid = (tid + k * NS) < NW
        safe = jnp.full((LANES,), TRASH << PACK_BITS, jnp.int32)

        @pl.loop(0, W, step=LANES)
        def _(j):
            pidxc[pl.ds(j, LANES)] = safe

        def comp_body(j, p_off):
            sv = srcb[p, pl.ds(j * LANES, LANES)]
            sv = jnp.where(sv >= N, sv - N, sv)
            sv = jnp.where(sv >= N, sv - N, sv)
            tv = tgtb[p, pl.ds(j * LANES, LANES)]
            rel = tv - lo
            own = (rel >= 0) & (rel < HALF_T) & valid
            packed = sv | jnp.where(own, rel, TRASH) << PACK_BITS
            plsc.store_compressed(pidxc.at[pl.ds(p_off, LANES)], packed,
                                  mask=own)
            return p_off + jnp.sum(own.astype(jnp.int32))

        cnt = lax.fori_loop(0, W // LANES, comp_body, 0)

        @pl.loop(0, MAXCH)
        def _(r):
            @pl.loop(0, GCHUNK, step=LANES)
            def _(q):
                v = pidxc[pl.ds(r * GCHUNK + q, LANES)]
                gidx3[r, pl.ds(q, LANES)] = v & PACK_MASK
                sidx3[r, pl.ds(q, LANES)] = v >> PACK_BITS

        def live(r):
            return r * GCHUNK < cnt

        for r in range(2):
            @pl.when(live(r))
            def _(r=r):
                pltpu.async_copy(x_hbm.at[gidx3.at[r]], rows[r & 1],
                                 sem_g[r & 1])

        for r in range(MAXCH):
            b = r & 1

            @pl.when(live(r))
            def _(r=r, b=b):
                pltpu.make_async_copy(x_hbm.at[gidx3.at[r]], rows[b],
                                      sem_g[b]).wait()
                pltpu.async_copy(rows[b], acc.at[sidx3.at[r]], sem_s[b],
                                 add=True)

            if r + 2 < MAXCH:
                @pl.when(live(r + 2))
                def _(r=r, b=b):
                    pltpu.make_async_copy(rows[b], acc.at[sidx3.at[r]],
                                          sem_s[b]).wait()
                    pltpu.async_copy(x_hbm.at[gidx3.at[r + 2]], rows[b],
                                     sem_g[b])

        for r in range(MAXCH):
            if r + 2 < MAXCH:
                pending = live(r) & jnp.logical_not(live(r + 2))
            else:
                pending = live(r)

            @pl.when(pending)
            def _(r=r):
                pltpu.make_async_copy(rows[r & 1], acc.at[sidx3.at[r]],
                                      sem_s[r & 1]).wait()

    start_idx_load(0, 0)

    @pl.loop(0, WIN_PER_TILE, step=2)
    def _(k):
        run_window(k, 0)
        run_window(k + 1, 1)

    plsc.subcore_barrier()

    pltpu.sync_copy(acc.at[pl.ds(zbase, WB_ROWS), :],
                    out_hbm.at[c, pl.ds(zbase, WB_ROWS), :])


def _sc_compiler_params():
    cp = pltpu.CompilerParams()
    fields = pltpu.CompilerParams.__dataclass_fields__
    if "needs_layout_passes" in fields:
        cp = dataclasses.replace(cp, needs_layout_passes=False)
    if "use_tc_tiling_on_sc" in fields:
        cp = dataclasses.replace(cp, use_tc_tiling_on_sc=False)
    return cp


@jax.jit
def _sc_scatter(x, src, tgt):
    mesh = plsc.VectorSubcoreMesh(core_axis_name="c", subcore_axis_name="s")
    kfn = pl.kernel(
        _sc_body,
        out_type=jax.ShapeDtypeStruct((NC, ACC_ROWS, D), jnp.float32),
        mesh=mesh,
        scratch_types=[
            pltpu.VMEM((2, W), jnp.int32),
            pltpu.VMEM((2, W), jnp.int32),
            pltpu.VMEM((W,), jnp.int32),
            pltpu.VMEM((MAXCH, GCHUNK), jnp.int32),
            pltpu.VMEM((MAXCH, GCHUNK), jnp.int32),
            pltpu.VMEM((GCHUNK, D), jnp.float32),
            pltpu.VMEM((GCHUNK, D), jnp.float32),
            pltpu.SemaphoreType.DMA,
            pltpu.SemaphoreType.DMA,
            pltpu.SemaphoreType.DMA,
            pltpu.SemaphoreType.DMA,
            pltpu.SemaphoreType.DMA,
            pltpu.SemaphoreType.DMA,
            pltpu.VMEM_SHARED((ACC_ROWS, D), jnp.float32),
        ],
        compiler_params=_sc_compiler_params(),
    )
    return kfn(x, src, tgt)


def _silu(v):
    return v / (1.0 + jnp.exp(-v))


def _mlp_body(x_ref, a0_ref, a1_ref, a2_ref,
              win_ref, bin_ref, wgs_ref, bgs_ref,
              w1a_ref, b1a_ref, w2a_ref, b2a_ref,
              w1b_ref, b1b_ref, w2b_ref, b2b_ref, out_ref):
    feats = jnp.concatenate(
        [x_ref[...], a0_ref[0], a1_ref[0], a2_ref[0]], axis=-1)
    h = _silu(jnp.dot(feats, win_ref[...],
                      preferred_element_type=jnp.float32) + bin_ref[...])
    gs = jnp.dot(feats, wgs_ref[...],
                 preferred_element_type=jnp.float32) + bgs_ref[...]
    for w1, b1, w2, b2 in ((w1a_ref, b1a_ref, w2a_ref, b2a_ref),
                           (w1b_ref, b1b_ref, w2b_ref, b2b_ref)):
        skip = h
        h = _silu(jnp.dot(h, w1[...],
                          preferred_element_type=jnp.float32) + b1[...])
        h = jnp.dot(h, w2[...], preferred_element_type=jnp.float32) + b2[...]
        h = h + skip
    out_ref[...] = h + gs


BLK = 1000
NBLK = N // BLK


def _hop1_map(i):
    cc = jnp.where(i >= 5, 1, 0)
    return cc, 10 + i - 15 * cc, 0


def _full(shape):
    return pl.BlockSpec(shape, lambda i: (0,) * len(shape))


@jax.jit
def _mlp(x, acc, W_in, b_in, W_gs, b_gs, W1a, b1a, W2a, b2a, W1b, b1b, W2b, b2b):
    hop_blk = (1, BLK, D)
    specs = [
        pl.BlockSpec((BLK, D), lambda i: (i, 0)),
        pl.BlockSpec(hop_blk, lambda i: (0, i, 0)),
        pl.BlockSpec(hop_blk, _hop1_map),
        pl.BlockSpec(hop_blk, lambda i: (1, 5 + i, 0)),
        _full((HOPS * D + D, D)), _full((1, D)),
        _full((HOPS * D + D, D)), _full((1, D)),
        _full((D, D)), _full((1, D)), _full((D, D)), _full((1, D)),
        _full((D, D)), _full((1, D)), _full((D, D)), _full((1, D)),
    ]
    return pl.pallas_call(
        _mlp_body,
        grid=(NBLK,),
        in_specs=specs,
        out_specs=pl.BlockSpec((BLK, D), lambda i: (i, 0)),
        out_shape=jax.ShapeDtypeStruct((N, D), jnp.float32),
    )(x, acc, acc, acc,
      W_in, b_in.reshape(1, D), W_gs, b_gs.reshape(1, D),
      W1a, b1a.reshape(1, D), W2a, b2a.reshape(1, D),
      W1b, b1b.reshape(1, D), W2b, b2b.reshape(1, D))


def kernel(x, target, src, W_in, b_in, W_gs, b_gs,
           W1a, b1a, W2a, b2a, W1b, b1b, W2b, b2b):
    acc = _sc_scatter(x, src, target)
    return _mlp(x, acc, W_in, b_in, W_gs, b_gs,
                W1a, b1a, W2a, b2a, W1b, b1b, W2b, b2b)


# speedup vs baseline: 6.0481x; 6.0481x over previous
import dataclasses
import functools

import jax
import jax.numpy as jnp
from jax import lax
from jax.experimental import pallas as pl
from jax.experimental.pallas import tpu as pltpu
from jax.experimental.pallas import tpu_sc as plsc

N = 10000
D = 128
HALF_D = D // 2
E = 320000
HOPS = 3

NC = 2
NS = 16
LANES = 16

GCHUNK = 32
NCHAIN = 4
SUPER = 512
GPS = SUPER // GCHUNK
NSUPER = E // SUPER
SUPER_PER_TILE = -(-NSUPER // NS)
IDX_ROWS = E // GCHUNK
TRASH = HOPS * N
ACC_ROWS = TRASH + 80
WB_ROWS = ACC_ROWS // NS


def _sc_body(xs_hbm, src_hbm, tgt_hbm, out_hbm,
             gidx2, sidx2, rows0, rows1, rows2, rows3,
             sem_i0, sem_i1, sem_g0, sem_g1, sem_g2, sem_g3,
             sem_s0, sem_s1, sem_s2, sem_s3, acc):
    c = lax.axis_index("c")
    tid = lax.axis_index("s")
    rows = (rows0, rows1, rows2, rows3)
    sem_i = (sem_i0, sem_i1)
    sem_g = (sem_g0, sem_g1, sem_g2, sem_g3)
    sem_s = (sem_s0, sem_s1, sem_s2, sem_s3)

    @pl.loop(0, GCHUNK)
    def _(i):
        @pl.loop(0, HALF_D, step=LANES)
        def _(j):
            rows0[i, pl.ds(j, LANES)] = jnp.zeros((LANES,), jnp.float32)

    zbase = tid * WB_ROWS
    NZ = WB_ROWS // GCHUNK

    @pl.loop(0, NZ)
    def _(i):
        pltpu.sync_copy(rows0, acc.at[pl.ds(zbase + i * GCHUNK, GCHUNK), :])

    pltpu.sync_copy(rows0.at[pl.ds(0, WB_ROWS - NZ * GCHUNK), :],
                    acc.at[pl.ds(zbase + NZ * GCHUNK, WB_ROWS - NZ * GCHUNK), :])

    plsc.subcore_barrier()

    def idx_base(s):
        sup = tid + s * NS
        return jnp.where(sup < NSUPER, sup * GPS, 0)

    def start_idx_load(s, p):
        b = idx_base(s)
        pltpu.async_copy(src_hbm.at[pl.ds(b, GPS), :], gidx2.at[p], sem_i[p])
        pltpu.async_copy(tgt_hbm.at[pl.ds(b, GPS), :], sidx2.at[p], sem_i[p])

    def wait_idx_load(s, p):
        b = idx_base(s)
        pltpu.make_async_copy(src_hbm.at[pl.ds(b, GPS), :], gidx2.at[p],
                              sem_i[p]).wait()
        pltpu.make_async_copy(tgt_hbm.at[pl.ds(b, GPS), :], sidx2.at[p],
                              sem_i[p]).wait()

    def run_super(s, p):
        valid = (tid + s * NS) < NSUPER
        wait_idx_load(s, p)

        @pl.when(s + 1 < SUPER_PER_TILE)
        def _():
            start_idx_load(s + 1, 1 - p)

        @pl.loop(0, GPS)
        def _(r):
            @pl.loop(0, GCHUNK, step=LANES)
            def _(j):
                sv = gidx2[p, r, pl.ds(j, LANES)]
                sv = jnp.where(sv >= N, sv - N, sv)
                sv = jnp.where(sv >= N, sv - N, sv)
                gidx2[p, r, pl.ds(j, LANES)] = sv
                tv = sidx2[p, r, pl.ds(j, LANES)]
                sidx2[p, r, pl.ds(j, LANES)] = jnp.where(valid, tv, TRASH)

        xsrc = xs_hbm.at[c]
        h_g = [pltpu.async_copy(xsrc.at[gidx2.at[p, b]], rows[b], sem_g[b])
               for b in range(NCHAIN)]
        h_s = [None] * NCHAIN
        for r in range(GPS):
            b = r % NCHAIN
            h_g[b].wait()
            h_s[b] = pltpu.async_copy(rows[b], acc.at[sidx2.at[p, r]],
                                      sem_s[b], add=True)
            if r + NCHAIN < GPS:
                h_s[b].wait()
                h_g[b] = pltpu.async_copy(
                    xsrc.at[gidx2.at[p, r + NCHAIN]], rows[b], sem_g[b])
        for b in range(NCHAIN):
            h_s[b].wait()

    start_idx_load(0, 0)

    @pl.loop(0, SUPER_PER_TILE, step=2)
    def _(s):
        run_super(s, 0)
        run_super(s + 1, 1)

    plsc.subcore_barrier()

    pltpu.sync_copy(acc.at[pl.ds(zbase, WB_ROWS), :],
                    out_hbm.at[c, pl.ds(zbase, WB_ROWS), :])


def _sc_compiler_params():
    cp = pltpu.CompilerParams()
    fields = pltpu.CompilerParams.__dataclass_fields__
    if "needs_layout_passes" in fields:
        cp = dataclasses.replace(cp, needs_layout_passes=False)
    if "use_tc_tiling_on_sc" in fields:
        cp = dataclasses.replace(cp, use_tc_tiling_on_sc=False)
    return cp


@jax.jit
def _sc_scatter(xs, src, tgt):
    mesh = plsc.VectorSubcoreMesh(core_axis_name="c", subcore_axis_name="s")
    kfn = pl.kernel(
        _sc_body,
        out_type=jax.ShapeDtypeStruct((NC, ACC_ROWS, HALF_D), jnp.float32),
        mesh=mesh,
        scratch_types=[
            pltpu.VMEM((2, GPS, GCHUNK), jnp.int32),
            pltpu.VMEM((2, GPS, GCHUNK), jnp.int32),
            pltpu.VMEM((GCHUNK, HALF_D), jnp.float32),
            pltpu.VMEM((GCHUNK, HALF_D), jnp.float32),
            pltpu.VMEM((GCHUNK, HALF_D), jnp.float32),
            pltpu.VMEM((GCHUNK, HALF_D), jnp.float32),
            pltpu.SemaphoreType.DMA,
            pltpu.SemaphoreType.DMA,
            pltpu.SemaphoreType.DMA,
            pltpu.SemaphoreType.DMA,
            pltpu.SemaphoreType.DMA,
            pltpu.SemaphoreType.DMA,
            pltpu.SemaphoreType.DMA,
            pltpu.SemaphoreType.DMA,
            pltpu.SemaphoreType.DMA,
            pltpu.SemaphoreType.DMA,
            pltpu.VMEM_SHARED((ACC_ROWS, HALF_D), jnp.float32),
        ],
        compiler_params=_sc_compiler_params(),
    )
    return kfn(xs, src.reshape(IDX_ROWS, GCHUNK), tgt.reshape(IDX_ROWS, GCHUNK))


def _silu(v):
    return v / (1.0 + jnp.exp(-v))


def _mlp_body(x_ref, a0l_ref, a0r_ref, a1l_ref, a1r_ref, a2l_ref, a2r_ref,
              win_ref, bin_ref, wgs_ref, bgs_ref,
              w1a_ref, b1a_ref, w2a_ref, b2a_ref,
              w1b_ref, b1b_ref, w2b_ref, b2b_ref, out_ref):
    feats = jnp.concatenate(
        [x_ref[...], a0l_ref[0], a0r_ref[0], a1l_ref[0], a1r_ref[0],
         a2l_ref[0], a2r_ref[0]], axis=-1)
    h = _silu(jnp.dot(feats, win_ref[...],
                      preferred_element_type=jnp.float32) + bin_ref[...])
    gs = jnp.dot(feats, wgs_ref[...],
                 preferred_element_type=jnp.float32) + bgs_ref[...]
    for w1, b1, w2, b2 in ((w1a_ref, b1a_ref, w2a_ref, b2a_ref),
                           (w1b_ref, b1b_ref, w2b_ref, b2b_ref)):
        skip = h
        h = _silu(jnp.dot(h, w1[...],
                          preferred_element_type=jnp.float32) + b1[...])
        h = jnp.dot(h, w2[...], preferred_element_type=jnp.float32) + b2[...]
        h = h + skip
    out_ref[...] = h + gs


BLK = 1000
NBLK = N // BLK
HOP_STRIDE = N // BLK


def _hop_spec(h, half):
    return pl.BlockSpec((1, BLK, HALF_D),
                        lambda i, h=h, half=half: (half, HOP_STRIDE * h + i, 0))


def _full(shape):
    return pl.BlockSpec(shape, lambda i: (0,) * len(shape))


@jax.jit
def _mlp(x, acc, W_in, b_in, W_gs, b_gs, W1a, b1a, W2a, b2a, W1b, b1b, W2b, b2b):
    specs = [
        pl.BlockSpec((BLK, D), lambda i: (i, 0)),
        _hop_spec(0, 0), _hop_spec(0, 1),
        _hop_spec(1, 0), _hop_spec(1, 1),
        _hop_spec(2, 0), _hop_spec(2, 1),
        _full((HOPS * D + D, D)), _full((1, D)),
        _full((HOPS * D + D, D)), _full((1, D)),
        _full((D, D)), _full((1, D)), _full((D, D)), _full((1, D)),
        _full((D, D)), _full((1, D)), _full((D, D)), _full((1, D)),
    ]
    return pl.pallas_call(
        _mlp_body,
        grid=(NBLK,),
        in_specs=specs,
        out_specs=pl.BlockSpec((BLK, D), lambda i: (i, 0)),
        out_shape=jax.ShapeDtypeStruct((N, D), jnp.float32),
    )(x, acc, acc, acc, acc, acc, acc,
      W_in, b_in.reshape(1, D), W_gs, b_gs.reshape(1, D),
      W1a, b1a.reshape(1, D), W2a, b2a.reshape(1, D),
      W1b, b1b.reshape(1, D), W2b, b2b.reshape(1, D))


def kernel(x, target, src, W_in, b_in, W_gs, b_gs,
           W1a, b1a, W2a, b2a, W1b, b1b, W2b, b2b):
    xs = jnp.stack([x[:, :HALF_D], x[:, HALF_D:]])
    acc = _sc_scatter(xs, src, target)
    return _mlp(x, acc, W_in, b_in, W_gs, b_gs,
                W1a, b1a, W2a, b2a, W1b, b1b, W2b, b2b)
